# baseline (speedup 1.0000x reference)
import functools

import jax
import jax.numpy as jnp
from jax import lax
from jax.experimental import pallas as pl
from jax.experimental.pallas import tpu as pltpu
from jax.experimental.pallas import tpu_sc as plsc

_VOCAB = 200
_HID = 1024
_PHID = 512
_OUT_W = 49152
_N_ROWS = 6400
_CB = 2048
_N_CB = _OUT_W // _CB


def _table_body(emb_ref, w1_ref, b1_ref, w2_ref, b2_ref, out_ref, h_ref):
    @pl.when(pl.program_id(0) == 0)
    def _():
        h = jnp.dot(emb_ref[...], w1_ref[...], preferred_element_type=jnp.float32)
        h_ref[...] = jnp.tanh(h + b1_ref[...])

    out_ref[...] = (
        jnp.dot(h_ref[...], w2_ref[...], preferred_element_type=jnp.float32)
        + b2_ref[...]
    )


def _compute_table(emb_table, W1, b1, W2, b2):
    return pl.pallas_call(
        _table_body,
        grid=(_N_CB,),
        in_specs=[
            pl.BlockSpec((_VOCAB, _HID), lambda j: (0, 0)),
            pl.BlockSpec((_HID, _PHID), lambda j: (0, 0)),
            pl.BlockSpec((1, _PHID), lambda j: (0, 0)),
            pl.BlockSpec((_PHID, _CB), lambda j: (0, j)),
            pl.BlockSpec((1, _CB), lambda j: (0, j)),
        ],
        out_specs=pl.BlockSpec((_VOCAB, _CB), lambda j: (0, j)),
        out_shape=jax.ShapeDtypeStruct((_VOCAB, _OUT_W), jnp.float32),
        scratch_shapes=[pltpu.VMEM((_VOCAB, _PHID), jnp.float32)],
    )(emb_table, W1, b1.reshape(1, _PHID), W2, b2.reshape(1, _OUT_W))


def _gather(table, idx_flat):
    info = plsc.get_sparse_core_info()
    nc, ns = info.num_cores, info.num_subcores
    nw = nc * ns
    rpw = _N_ROWS // nw
    pairs = rpw // 2
    idx_pairs = idx_flat.reshape(nw, pairs, 2)

    mesh = plsc.VectorSubcoreMesh(core_axis_name="c", subcore_axis_name="s")

    @functools.partial(
        pl.kernel,
        mesh=mesh,
        out_type=jax.ShapeDtypeStruct((_N_ROWS // 2, 2, _OUT_W), jnp.float32),
        scratch_types=[
            pltpu.VMEM((pairs, 2), jnp.int32),
            pltpu.VMEM((2, _OUT_W), jnp.float32),
            pltpu.SemaphoreType.DMA,
            pltpu.SemaphoreType.DMA,
        ],
    )
    def gather_k(table_hbm, idx_hbm, out_hbm, idx_v, buf_v, sem_g, sem_w):
        wid = lax.axis_index("s") * nc + lax.axis_index("c")
        pair_base = wid * pairs
        pltpu.sync_copy(idx_hbm.at[wid], idx_v)

        def body(p, carry):
            pltpu.async_copy(table_hbm.at[idx_v.at[p]], buf_v, sem_g).wait()
            pltpu.async_copy(buf_v, out_hbm.at[pair_base + p], sem_w).wait()
            return carry

        lax.fori_loop(0, pairs, body, 0)

    return gather_k(table, idx_pairs)


def kernel(emb_table, W1, b1, W2, b2, prefix):
    table = _compute_table(emb_table, W1, b1, W2, b2)
    out = _gather(table, prefix.astype(jnp.int32).reshape(-1))
    return out.reshape(prefix.shape[0], prefix.shape[1], _OUT_W)


# speedup vs baseline: 1.7750x; 1.7750x over previous
import functools

import jax
import jax.numpy as jnp
from jax import lax
from jax.experimental import pallas as pl
from jax.experimental.pallas import tpu as pltpu
from jax.experimental.pallas import tpu_sc as plsc

_VOCAB = 200
_HID = 1024
_PHID = 512
_OUT_W = 49152
_N_ROWS = 6400
_CB = 2048
_N_CB = _OUT_W // _CB


def _table_body(emb_ref, w1_ref, b1_ref, w2_ref, b2_ref, out_ref, h_ref):
    @pl.when(pl.program_id(0) == 0)
    def _():
        h = jnp.dot(emb_ref[...], w1_ref[...], preferred_element_type=jnp.float32)
        h_ref[...] = jnp.tanh(h + b1_ref[...])

    out_ref[...] = (
        jnp.dot(h_ref[...], w2_ref[...], preferred_element_type=jnp.float32)
        + b2_ref[...]
    )


def _compute_table(emb_table, W1, b1, W2, b2):
    return pl.pallas_call(
        _table_body,
        grid=(_N_CB,),
        in_specs=[
            pl.BlockSpec((_VOCAB, _HID), lambda j: (0, 0)),
            pl.BlockSpec((_HID, _PHID), lambda j: (0, 0)),
            pl.BlockSpec((1, _PHID), lambda j: (0, 0)),
            pl.BlockSpec((_PHID, _CB), lambda j: (0, j)),
            pl.BlockSpec((1, _CB), lambda j: (0, j)),
        ],
        out_specs=pl.BlockSpec((_VOCAB, _CB), lambda j: (0, j)),
        out_shape=jax.ShapeDtypeStruct((_VOCAB, _OUT_W), jnp.float32),
        scratch_shapes=[pltpu.VMEM((_VOCAB, _PHID), jnp.float32)],
    )(emb_table, W1, b1.reshape(1, _PHID), W2, b2.reshape(1, _OUT_W))


def _gather(table, idx_flat):
    info = plsc.get_sparse_core_info()
    nc, ns = info.num_cores, info.num_subcores
    nw = nc * ns
    rpw = _N_ROWS // nw
    half = rpw // 2
    idx3 = idx_flat.reshape(nw, half, 2)

    mesh = plsc.VectorSubcoreMesh(core_axis_name="c", subcore_axis_name="s")

    @functools.partial(
        pl.kernel,
        mesh=mesh,
        out_type=jax.ShapeDtypeStruct((_N_ROWS, 1, _OUT_W), jnp.float32),
        scratch_types=[
            pltpu.VMEM((half, 2), jnp.int32),
            pltpu.VMEM((1, 1, _OUT_W), jnp.float32),
            pltpu.VMEM((1, 1, _OUT_W), jnp.float32),
            pltpu.SemaphoreType.DMA,
            pltpu.SemaphoreType.DMA,
            pltpu.SemaphoreType.DMA,
            pltpu.SemaphoreType.DMA,
        ],
    )
    def gather_k(table_hbm, idx_hbm, out_hbm, idx_v, buf_a, buf_b, sga, sgb, swa, swb):
        wid = lax.axis_index("s") * nc + lax.axis_index("c")
        base = wid * rpw
        pltpu.sync_copy(idx_hbm.at[wid], idx_v)

        def row_idx(j, par):
            return idx_v.at[j, pl.ds(par, 1)]

        def out_row(r):
            return out_hbm.at[pl.ds(r, 1)]

        pltpu.async_copy(table_hbm.at[row_idx(0, 0)], buf_a, sga)
        pltpu.async_copy(table_hbm.at[row_idx(0, 1)], buf_b, sgb)

        def body(j, carry):
            r0 = base + 2 * j
            pltpu.make_async_copy(table_hbm.at[row_idx(j, 0)], buf_a, sga).wait()
            pltpu.async_copy(buf_a, out_row(r0), swa)
            pltpu.make_async_copy(table_hbm.at[row_idx(j, 1)], buf_b, sgb).wait()
            pltpu.async_copy(buf_b, out_row(r0 + 1), swb)

            @pl.when(j + 1 < half)
            def _():
                pltpu.make_async_copy(buf_a, out_row(r0), swa).wait()
                pltpu.async_copy(table_hbm.at[row_idx(j + 1, 0)], buf_a, sga)
                pltpu.make_async_copy(buf_b, out_row(r0 + 1), swb).wait()
                pltpu.async_copy(table_hbm.at[row_idx(j + 1, 1)], buf_b, sgb)

            return carry

        lax.fori_loop(0, half, body, 0)
        pltpu.make_async_copy(buf_a, out_row(base + rpw - 2), swa).wait()
        pltpu.make_async_copy(buf_b, out_row(base + rpw - 1), swb).wait()

    return gather_k(table.reshape(_VOCAB, 1, _OUT_W), idx3)


def kernel(emb_table, W1, b1, W2, b2, prefix):
    table = _compute_table(emb_table, W1, b1, W2, b2)
    out = _gather(table, prefix.astype(jnp.int32).reshape(-1))
    return out.reshape(prefix.shape[0], prefix.shape[1], _OUT_W)


# speedup vs baseline: 1.7892x; 1.0080x over previous
import functools

import jax
import jax.numpy as jnp
from jax import lax
from jax.experimental import pallas as pl
from jax.experimental.pallas import tpu as pltpu
from jax.experimental.pallas import tpu_sc as plsc

_VOCAB = 200
_HID = 1024
_PHID = 512
_OUT_W = 49152
_SPLIT = 32
_SW = _OUT_W // _SPLIT
_QROWS = 8
_NQ = _SPLIT // _QROWS
_N_ROWS = 6400
_CB = 2048
_N_CB = _OUT_W // _CB
_DEPTH = 6
_LEAD = 3


def _h_body(emb_ref, w1_ref, b1_ref, h_ref):
    h = jnp.dot(emb_ref[...], w1_ref[...], preferred_element_type=jnp.float32)
    h_ref[...] = jnp.tanh(h + b1_ref[...])


def _proj_body(h_ref, w2_ref, b2_ref, out_ref):
    out_ref[...] = (
        jnp.dot(h_ref[...], w2_ref[...], preferred_element_type=jnp.float32)
        + b2_ref[...]
    )


def _compute_table(emb_table, W1, b1, W2, b2):
    h = pl.pallas_call(
        _h_body,
        out_shape=jax.ShapeDtypeStruct((_VOCAB, _PHID), jnp.float32),
    )(emb_table, W1, b1.reshape(1, _PHID))

    return pl.pallas_call(
        _proj_body,
        grid=(_N_CB,),
        in_specs=[
            pl.BlockSpec((_VOCAB, _PHID), lambda j: (0, 0)),
            pl.BlockSpec((_PHID, _CB), lambda j: (0, j)),
            pl.BlockSpec((1, _CB), lambda j: (0, j)),
        ],
        out_specs=pl.BlockSpec((_VOCAB, _CB), lambda j: (0, j)),
        out_shape=jax.ShapeDtypeStruct((_VOCAB, _OUT_W), jnp.float32),
    )(h, W2, b2.reshape(1, _OUT_W))


def _gather(table, idx_flat):
    info = plsc.get_sparse_core_info()
    nc, ns = info.num_cores, info.num_subcores
    nw = nc * ns
    rpw = _N_ROWS // nw
    items = rpw * _NQ

    sub = jnp.arange(_SPLIT, dtype=jnp.int32)
    qidx = (_SPLIT * idx_flat[:, None] + sub).reshape(nw, rpw, _SPLIT)

    mesh = plsc.VectorSubcoreMesh(core_axis_name="c", subcore_axis_name="s")

    @functools.partial(
        pl.kernel,
        mesh=mesh,
        out_type=jax.ShapeDtypeStruct((_SPLIT * _N_ROWS, _SW), jnp.float32),
        scratch_types=[
            pltpu.VMEM((rpw, _SPLIT), jnp.int32),
            pltpu.VMEM((2, _NQ, _QROWS, _SW), jnp.float32),
            pltpu.SemaphoreType.DMA((2, _NQ)),
            pltpu.SemaphoreType.DMA((2, _NQ)),
        ],
    )
    def gather_k(tab_hbm, qidx_hbm, out_hbm, qidx_v, bufs, sg, sw):
        wid = lax.axis_index("s") * nc + lax.axis_index("c")
        pltpu.sync_copy(qidx_hbm.at[wid], qidx_v)
        row0 = wid * rpw

        def gath(r, q):
            par = lax.rem(r, 2)
            idx_sl = qidx_v.at[r, pl.ds(_QROWS * q, _QROWS)]
            return pltpu.make_async_copy(
                tab_hbm.at[idx_sl], bufs.at[par, q], sg.at[par, q])

        def scat(r, q):
            par = lax.rem(r, 2)
            off = pl.multiple_of(_SPLIT * (row0 + r) + _QROWS * q, _QROWS)
            return pltpu.make_async_copy(
                bufs.at[par, q], out_hbm.at[pl.ds(off, _QROWS)],
                sw.at[par, q])

        for q in range(_NQ):
            gath(0, q).start()

        def body(r, carry):
            @pl.when(r >= 1)
            def _():
                for q in range(_NQ):
                    scat(r - 1, q).wait()

            @pl.when(r + 1 < rpw)
            def _():
                for q in range(_NQ):
                    gath(r + 1, q).start()

            for q in range(_NQ):
                gath(r, q).wait()
                scat(r, q).start()

            return carry

        lax.fori_loop(0, rpw, body, 0)

        for q in range(_NQ):
            scat(rpw - 1, q).wait()

    return gather_k(table.reshape(_SPLIT * _VOCAB, _SW), qidx)


def kernel(emb_table, W1, b1, W2, b2, prefix):
    table = _compute_table(emb_table, W1, b1, W2, b2)
    out = _gather(table, prefix.astype(jnp.int32).reshape(-1))
    return out.reshape(prefix.shape[0], prefix.shape[1], _OUT_W)


# speedup vs baseline: 2.7655x; 1.5457x over previous
import functools

import jax
import jax.numpy as jnp
from jax import lax
from jax.experimental import pallas as pl
from jax.experimental.pallas import tpu as pltpu
from jax.experimental.pallas import tpu_sc as plsc

_VOCAB = 200
_HID = 1024
_PHID = 512
_OUT_W = 49152
_N_ROWS = 6400
_CB = 2048
_N_CB = _OUT_W // _CB
_CW = 2048
_NC_CH = _OUT_W // _CW
_G = 8


def _h_body(emb_ref, w1_ref, b1_ref, h_ref):
    h = jnp.dot(emb_ref[...], w1_ref[...], preferred_element_type=jnp.float32)
    h_ref[...] = jnp.tanh(h + b1_ref[...])


def _proj_body(h_ref, w2_ref, b2_ref, out_ref):
    out_ref[...] = (
        jnp.dot(h_ref[...], w2_ref[...], preferred_element_type=jnp.float32)
        + b2_ref[...]
    )


def _compute_table(emb_table, W1, b1, W2, b2):
    h = pl.pallas_call(
        _h_body,
        out_shape=jax.ShapeDtypeStruct((_VOCAB, _PHID), jnp.float32),
    )(emb_table, W1, b1.reshape(1, _PHID))

    return pl.pallas_call(
        _proj_body,
        grid=(_N_CB,),
        in_specs=[
            pl.BlockSpec((_VOCAB, _PHID), lambda j: (0, 0)),
            pl.BlockSpec((_PHID, _CB), lambda j: (0, j)),
            pl.BlockSpec((1, _CB), lambda j: (0, j)),
        ],
        out_specs=pl.BlockSpec((_VOCAB, _CB), lambda j: (0, j)),
        out_shape=jax.ShapeDtypeStruct((_VOCAB, _OUT_W), jnp.float32),
    )(h, W2, b2.reshape(1, _OUT_W))


def _gather(table, idx_flat, B, P):
    info = plsc.get_sparse_core_info()
    nc, ns = info.num_cores, info.num_subcores
    nw = nc * ns
    rpw = _N_ROWS // nw
    ngr = rpw // _G
    idx3 = idx_flat.reshape(nw, ngr, _G)

    mesh = plsc.VectorSubcoreMesh(core_axis_name="c", subcore_axis_name="s")

    @functools.partial(
        pl.kernel,
        mesh=mesh,
        out_type=jax.ShapeDtypeStruct((B, P, _OUT_W), jnp.float32),
        scratch_types=[
            pltpu.VMEM((ngr, _G), jnp.int32),
            pltpu.VMEM((_G, _CW), jnp.float32),
            pltpu.SemaphoreType.DMA,
            pltpu.SemaphoreType.DMA,
        ],
    )
    def gather_k(tab_hbm, idx_hbm, out_hbm, idx_v, buf, sga, swa):
        wid = lax.axis_index("s") * nc + lax.axis_index("c")
        pltpu.sync_copy(idx_hbm.at[wid], idx_v)
        row0 = wid * rpw

        def body(it, carry):
            k = lax.div(it, _NC_CH)
            c = lax.rem(it, _NC_CH)
            coff = pl.multiple_of(_CW * c, _CW)
            pltpu.async_copy(
                tab_hbm.at[idx_v.at[k], pl.ds(coff, _CW)], buf, sga).wait()
            grow = row0 + _G * k
            b = lax.div(grow, P)
            p = pl.multiple_of(lax.rem(grow, P), _G)
            pltpu.async_copy(
                buf, out_hbm.at[b, pl.ds(p, _G), pl.ds(coff, _CW)],
                swa).wait()
            return carry

        lax.fori_loop(0, ngr * _NC_CH, body, 0)

    return gather_k(table, idx3)


def kernel(emb_table, W1, b1, W2, b2, prefix):
    B, P = prefix.shape
    table = _compute_table(emb_table, W1, b1, W2, b2)
    return _gather(table, prefix.astype(jnp.int32).reshape(-1), B, P)


# speedup vs baseline: 4.1237x; 1.4911x over previous
import functools

import jax
import jax.numpy as jnp
from jax import lax
from jax.experimental import pallas as pl
from jax.experimental.pallas import tpu as pltpu
from jax.experimental.pallas import tpu_sc as plsc

_VOCAB = 200
_HID = 1024
_PHID = 512
_OUT_W = 49152
_N_ROWS = 6400
_CB = 2048
_N_CB = _OUT_W // _CB
_CW = 2048
_NC_CH = _OUT_W // _CW
_G = 8
_RING = 4
_LEAD = 2


def _h_body(emb_ref, w1_ref, b1_ref, h_ref):
    h = jnp.dot(emb_ref[...], w1_ref[...], preferred_element_type=jnp.float32)
    h_ref[...] = jnp.tanh(h + b1_ref[...])


def _proj_body(h_ref, w2_ref, b2_ref, out_ref):
    out_ref[...] = (
        jnp.dot(h_ref[...], w2_ref[...], preferred_element_type=jnp.float32)
        + b2_ref[...]
    )


def _compute_table(emb_table, W1, b1, W2, b2):
    h = pl.pallas_call(
        _h_body,
        out_shape=jax.ShapeDtypeStruct((_VOCAB, _PHID), jnp.float32),
    )(emb_table, W1, b1.reshape(1, _PHID))

    return pl.pallas_call(
        _proj_body,
        grid=(_N_CB,),
        in_specs=[
            pl.BlockSpec((_VOCAB, _PHID), lambda j: (0, 0)),
            pl.BlockSpec((_PHID, _CB), lambda j: (0, j)),
            pl.BlockSpec((1, _CB), lambda j: (0, j)),
        ],
        out_specs=pl.BlockSpec((_VOCAB, _CB), lambda j: (0, j)),
        out_shape=jax.ShapeDtypeStruct((_VOCAB, _OUT_W), jnp.float32),
    )(h, W2, b2.reshape(1, _OUT_W))


def _gather(table, idx_flat, B, P):
    info = plsc.get_sparse_core_info()
    nc, ns = info.num_cores, info.num_subcores
    nw = nc * ns
    rpw = _N_ROWS // nw
    ngr = rpw // _G
    idx3 = idx_flat.reshape(nw, ngr, _G)

    mesh = plsc.VectorSubcoreMesh(core_axis_name="c", subcore_axis_name="s")

    @functools.partial(
        pl.kernel,
        mesh=mesh,
        out_type=jax.ShapeDtypeStruct((B, P, _OUT_W), jnp.float32),
        scratch_types=[
            pltpu.VMEM((ngr, _G), jnp.int32),
            pltpu.VMEM((_RING, _G, _CW), jnp.float32),
            pltpu.SemaphoreType.DMA((_RING,)),
            pltpu.SemaphoreType.DMA((_RING,)),
        ],
    )
    def gather_k(tab_hbm, idx_hbm, out_hbm, idx_v, bufs, sg, sw):
        wid = lax.axis_index("s") * nc + lax.axis_index("c")
        pltpu.sync_copy(idx_hbm.at[wid], idx_v)
        row0 = wid * rpw
        nit = ngr * _NC_CH

        def slot(it):
            return lax.rem(it, _RING)

        def gath(it):
            k = lax.div(it, _NC_CH)
            c = lax.rem(it, _NC_CH)
            coff = pl.multiple_of(_CW * c, _CW)
            return pltpu.make_async_copy(
                tab_hbm.at[idx_v.at[k], pl.ds(coff, _CW)],
                bufs.at[slot(it)], sg.at[slot(it)])

        def scat(it):
            k = lax.div(it, _NC_CH)
            c = lax.rem(it, _NC_CH)
            coff = pl.multiple_of(_CW * c, _CW)
            grow = row0 + _G * k
            b = lax.div(grow, P)
            p = pl.multiple_of(lax.rem(grow, P), _G)
            return pltpu.make_async_copy(
                bufs.at[slot(it)],
                out_hbm.at[b, pl.ds(p, _G), pl.ds(coff, _CW)],
                sw.at[slot(it)])

        def prologue(i, carry):
            gath(i).start()
            return carry

        lax.fori_loop(0, _LEAD, prologue, 0)

        def body(it, carry):
            @pl.when(it >= _LEAD)
            def _():
                scat(it - _LEAD).wait()

            @pl.when(it + _LEAD < nit)
            def _():
                gath(it + _LEAD).start()

            gath(it).wait()
            scat(it).start()
            return carry

        lax.fori_loop(0, nit, body, 0)

        def drain(t, carry):
            scat(nit - _LEAD + t).wait()
            return carry

        lax.fori_loop(0, _LEAD, drain, 0)

    return gather_k(table, idx3)


def kernel(emb_table, W1, b1, W2, b2, prefix):
    B, P = prefix.shape
    table = _compute_table(emb_table, W1, b1, W2, b2)
    return _gather(table, prefix.astype(jnp.int32).reshape(-1), B, P)
